# baseline (speedup 1.0000x reference)
import functools

import jax
import jax.numpy as jnp
from jax import lax
from jax.experimental import pallas as pl
from jax.experimental.pallas import tpu as pltpu
from jax.experimental.pallas import tpu_sc as plsc

NC = 2
NS = 16
NW = NC * NS
NVT = 4
SEQ = 50
NBT = 8


@jax.jit
def _sc_bigram_lookup(xt, table):
    mesh = plsc.VectorSubcoreMesh(core_axis_name="c", subcore_axis_name="s")

    @functools.partial(
        pl.kernel,
        mesh=mesh,
        compiler_params=pltpu.CompilerParams(
            use_tc_tiling_on_sc=False, needs_layout_passes=False
        ),
        out_type=jax.ShapeDtypeStruct((SEQ, 125, NBT, 8, 128), jnp.float32),
        scratch_types=[
            pltpu.VMEM((1000, 40), jnp.float32),
            pltpu.VMEM((1024,), jnp.int32),
            pltpu.VMEM((1024,), jnp.int32),
            pltpu.VMEM((NVT, NBT, 8, 128), jnp.float32),
            pltpu.VMEM((NVT, NBT, 8, 128), jnp.float32),
            pltpu.SemaphoreType.DMA,
            pltpu.SemaphoreType.DMA,
            pltpu.SemaphoreType.DMA,
            pltpu.SemaphoreType.DMA,
        ],
    )
    def run(
        xt_hbm, table_hbm, out_hbm,
        tslice, idx0, idx1, strip0, strip1, isem0, isem1, ssem0, ssem1,
    ):
        w = lax.axis_index("s") * NC + lax.axis_index("c")
        vt0 = jnp.minimum(NVT * w, 125 - NVT)
        c0 = vt0 * 8
        idxs = (idx0, idx1)
        strips = (strip0, strip1)
        isems = (isem0, isem1)
        ssems = (ssem0, ssem1)

        pltpu.async_copy(xt_hbm.at[0], idx0, isem0)
        pltpu.sync_copy(
            table_hbm.at[:, pl.ds(c0, 8 * NVT)], tslice.at[:, pl.ds(0, 8 * NVT)]
        )

        def plane_body(s2, carry):
            for a in range(2):
                s = s2 * 2 + a
                pltpu.make_async_copy(xt_hbm.at[s], idxs[a], isems[a]).wait()

                @pl.when(s + 1 < SEQ)
                def _():
                    pltpu.async_copy(xt_hbm.at[s + 1], idxs[1 - a], isems[1 - a])

                @pl.when(s >= 2)
                def _():
                    pltpu.make_async_copy(
                        strips[a], out_hbm.at[0, pl.ds(0, NVT)], ssems[a]
                    ).wait()

                idx = idxs[a]
                strip = strips[a]

                @plsc.parallel_loop(0, NBT * 8, unroll=32)
                def tb_body(i):
                    bt = i // 8
                    q = i % 8
                    tokens = idx[pl.ds(i * 16, 16)]
                    cols = []
                    for vtl in range(NVT):
                        for vr in range(8):
                            vcol = jnp.full((16,), vtl * 8 + vr, jnp.int32)
                            cols.append(
                                (vtl, vr, plsc.load_gather(tslice, [tokens, vcol]))
                            )
                    for vtl, vr, col in cols:
                        strip[vtl, bt, vr, pl.ds(q * 16, 16)] = col

                pltpu.async_copy(
                    strip, out_hbm.at[s, pl.ds(vt0, NVT)], ssems[a]
                )
            return carry

        lax.fori_loop(0, SEQ // 2, plane_body, 0)
        for a in range(2):
            pltpu.make_async_copy(
                strips[a], out_hbm.at[0, pl.ds(0, NVT)], ssems[a]
            ).wait()

    return run(xt, table)


def kernel(x, token_table):
    batch, seq = x.shape
    xt = x.T.astype(jnp.int32)
    out5d = _sc_bigram_lookup(xt, token_table)
    return out5d.transpose(2, 4, 0, 1, 3).reshape(batch, seq, token_table.shape[1])


# speedup vs baseline: 1.3019x; 1.3019x over previous
import functools

import jax
import jax.numpy as jnp
from jax import lax
from jax.experimental import pallas as pl
from jax.experimental.pallas import tpu as pltpu
from jax.experimental.pallas import tpu_sc as plsc

NC = 2
NS = 16
NW = NC * NS
NVT = 4
SEQ = 50
NBT = 8


@jax.jit
def _sc_bigram_lookup(xt, table):
    mesh = plsc.VectorSubcoreMesh(core_axis_name="c", subcore_axis_name="s")

    @functools.partial(
        pl.kernel,
        mesh=mesh,
        compiler_params=pltpu.CompilerParams(
            use_tc_tiling_on_sc=False, needs_layout_passes=False
        ),
        out_type=jax.ShapeDtypeStruct((SEQ, 125, NBT, 8, 128), jnp.float32),
        scratch_types=[
            pltpu.VMEM((1000, 40), jnp.float32),
            pltpu.VMEM((1024,), jnp.int32),
            pltpu.VMEM((1024,), jnp.int32),
            pltpu.VMEM((NVT, NBT, 8, 128), jnp.float32),
            pltpu.VMEM((NVT, NBT, 8, 128), jnp.float32),
            pltpu.SemaphoreType.DMA,
            pltpu.SemaphoreType.DMA,
            pltpu.SemaphoreType.DMA,
            pltpu.SemaphoreType.DMA,
        ],
    )
    def run(
        xt_hbm, table_hbm, out_hbm,
        tslice, idx0, idx1, strip0, strip1, isem0, isem1, ssem0, ssem1,
    ):
        w = lax.axis_index("s") * NC + lax.axis_index("c")
        vt0 = jnp.minimum(NVT * w, 125 - NVT)
        c0 = vt0 * 8
        idxs = (idx0, idx1)
        strips = (strip0, strip1)
        isems = (isem0, isem1)
        ssems = (ssem0, ssem1)

        pltpu.async_copy(xt_hbm.at[0], idx0, isem0)
        pltpu.sync_copy(
            table_hbm.at[:, pl.ds(c0, 8 * NVT)], tslice.at[:, pl.ds(0, 8 * NVT)]
        )

        def plane_body(s2, carry):
            for a in range(2):
                s = s2 * 2 + a
                pltpu.make_async_copy(xt_hbm.at[s], idxs[a], isems[a]).wait()

                @pl.when(s + 1 < SEQ)
                def _():
                    pltpu.async_copy(xt_hbm.at[s + 1], idxs[1 - a], isems[1 - a])

                @pl.when(s >= 2)
                def _():
                    pltpu.make_async_copy(
                        strips[a], out_hbm.at[0, pl.ds(0, NVT)], ssems[a]
                    ).wait()

                idx = idxs[a]
                strip = strips[a]

                @plsc.parallel_loop(0, NBT * 8, unroll=16)
                def tb_body(i):
                    bt = i // 8
                    q = i % 8
                    tokens = idx[pl.ds(i * 16, 16)]
                    cols = []
                    for vtl in range(NVT):
                        for vr in range(8):
                            vcol = jnp.full((16,), vtl * 8 + vr, jnp.int32)
                            cols.append(
                                (vtl, vr, plsc.load_gather(tslice, [tokens, vcol]))
                            )
                    for vtl, vr, col in cols:
                        strip[vtl, bt, vr, pl.ds(q * 16, 16)] = col

                pltpu.async_copy(
                    strip, out_hbm.at[s, pl.ds(vt0, NVT)], ssems[a]
                )
            return carry

        lax.fori_loop(0, SEQ // 2, plane_body, 0)
        for a in range(2):
            pltpu.make_async_copy(
                strips[a], out_hbm.at[0, pl.ds(0, NVT)], ssems[a]
            ).wait()

    return run(xt, table)


def kernel(x, token_table):
    batch, seq = x.shape
    xt = x.T.astype(jnp.int32)
    out5d = _sc_bigram_lookup(xt, token_table)
    return out5d.transpose(2, 4, 0, 1, 3).reshape(batch, seq, token_table.shape[1])
